# Initial kernel scaffold; baseline (speedup 1.0000x reference)
import jax
import jax.numpy as jnp
from jax.experimental import pallas as pl


def kernel(feat, edge_index, W_neigh, b_neigh, W_self, b_self):
    raise NotImplementedError("write your pallas kernel here")


# speedup vs baseline: 5.5487x; 5.5487x over previous
import functools

import jax
import jax.numpy as jnp
from jax import lax
from jax.experimental import pallas as pl
from jax.experimental.pallas import tpu as pltpu
from jax.experimental.pallas import tpu_sc as plsc

N_NODES = 10000
N_EDGES = 320000
D_IN = 128
D_OUT = 128

NC = 2
NS = 16
NW = NC * NS

WIDE = 144
CHUNK = 128
E_PAD = ((N_EDGES + NW * CHUNK - 1) // (NW * CHUNK)) * (NW * CHUNK)
EDGES_PER_W = E_PAD // NW
CHUNKS_PER_W = EDGES_PER_W // CHUNK
ACC_ROWS = ((N_NODES + NS * 8 - 1) // (NS * 8)) * (NS * 8)
ROWS_PER_TILE = ACC_ROWS // NS


def _sc_aggregate():
    mesh = plsc.VectorSubcoreMesh(
        core_axis_name="c", subcore_axis_name="s", num_cores=NC, num_subcores=NS)

    @functools.partial(
        pl.kernel,
        out_type=jax.ShapeDtypeStruct((NC, ACC_ROWS, WIDE), jnp.float32),
        mesh=mesh,
        scratch_types=[
            pltpu.VMEM((CHUNK,), jnp.int32),
            pltpu.VMEM((CHUNK,), jnp.int32),
            pltpu.VMEM((CHUNK, WIDE), jnp.float32),
            pltpu.VMEM_SHARED((ACC_ROWS, WIDE), jnp.float32),
            pltpu.SemaphoreType.DMA,
        ],
        compiler_params=pltpu.CompilerParams(use_tc_tiling_on_sc=False),
    )
    def agg(feat_hbm, src_hbm, dst_hbm, zeros_hbm, out_hbm, sidx, didx, rows, acc, sem):
        c = lax.axis_index("c")
        s = lax.axis_index("s")
        wid = s * NC + c
        pltpu.sync_copy(zeros_hbm, acc.at[pl.ds(s * ROWS_PER_TILE, ROWS_PER_TILE)])
        plsc.subcore_barrier()

        base = wid * EDGES_PER_W

        def body(i, _):
            off = base + i * CHUNK
            pltpu.sync_copy(src_hbm.at[pl.ds(off, CHUNK)], sidx)
            pltpu.sync_copy(dst_hbm.at[pl.ds(off, CHUNK)], didx)
            pltpu.async_copy(feat_hbm.at[sidx], rows, sem).wait()
            pltpu.sync_copy(rows, acc.at[didx], add=True)
            return ()

        lax.fori_loop(0, CHUNKS_PER_W, body, ())
        plsc.subcore_barrier()
        pltpu.sync_copy(
            acc.at[pl.ds(s * ROWS_PER_TILE, ROWS_PER_TILE)],
            out_hbm.at[c].at[pl.ds(s * ROWS_PER_TILE, ROWS_PER_TILE)],
        )

    return agg


_SC_AGG = _sc_aggregate()

_BLK = 400
_GRID = N_NODES // _BLK


def _tc_body(feat_ref, p0_ref, p1_ref, wn_ref, ws_ref, b_ref, o_ref):
    acc = p0_ref[...] + p1_ref[...]
    nsum = acc[:, :D_IN]
    deg = acc[:, D_IN:D_IN + 1]
    scale = 1.0 / jnp.maximum(deg, 1.0)
    h = jnp.dot(feat_ref[...], ws_ref[...], preferred_element_type=jnp.float32)
    h = h + jnp.dot(nsum * scale, wn_ref[...], preferred_element_type=jnp.float32)
    o_ref[...] = h + b_ref[...]


def _tc_combine(feat, p0, p1, w_neigh, w_self, bias):
    return pl.pallas_call(
        _tc_body,
        grid=(_GRID,),
        in_specs=[
            pl.BlockSpec((_BLK, D_IN), lambda i: (i, 0)),
            pl.BlockSpec((_BLK, WIDE), lambda i: (i, 0)),
            pl.BlockSpec((_BLK, WIDE), lambda i: (i, 0)),
            pl.BlockSpec((D_IN, D_OUT), lambda i: (0, 0)),
            pl.BlockSpec((D_IN, D_OUT), lambda i: (0, 0)),
            pl.BlockSpec((1, D_OUT), lambda i: (0, 0)),
        ],
        out_specs=pl.BlockSpec((_BLK, D_OUT), lambda i: (i, 0)),
        out_shape=jax.ShapeDtypeStruct((N_NODES, D_OUT), jnp.float32),
    )(feat, p0, p1, w_neigh, w_self, bias)


def kernel(feat, edge_index, W_neigh, b_neigh, W_self, b_self):
    feat_aug = jnp.pad(feat, ((0, 0), (0, WIDE - D_IN)))
    feat_aug = feat_aug.at[:, D_IN].set(1.0)

    pad = E_PAD - N_EDGES
    spread = jnp.arange(pad, dtype=jnp.int32)
    src = jnp.concatenate([edge_index[0], spread % N_NODES])
    dst = jnp.concatenate(
        [edge_index[1], N_NODES + spread % (ACC_ROWS - N_NODES)])

    zeros = jnp.zeros((ROWS_PER_TILE, WIDE), dtype=jnp.float32)
    parts = _SC_AGG(feat_aug, src, dst, zeros)
    bias = (b_neigh + b_self).reshape(1, D_OUT)
    return _tc_combine(feat, parts[0], parts[1], W_neigh, W_self, bias)


# speedup vs baseline: 8.4657x; 1.5257x over previous
import functools

import jax
import jax.numpy as jnp
from jax import lax
from jax.experimental import pallas as pl
from jax.experimental.pallas import tpu as pltpu
from jax.experimental.pallas import tpu_sc as plsc

N_NODES = 10000
N_EDGES = 320000
D_IN = 128
D_OUT = 128

NC = 2
NS = 16
NW = NC * NS

WIDE = 144
CHUNK = 128
IB = 8
E_PAD = ((N_EDGES + NW * CHUNK * IB - 1) // (NW * CHUNK * IB)) * (NW * CHUNK * IB)
EDGES_PER_W = E_PAD // NW
CHUNKS_PER_W = EDGES_PER_W // CHUNK
BLOCKS_PER_W = CHUNKS_PER_W // IB
ACC_ROWS = ((N_NODES + NS * 8 - 1) // (NS * 8)) * (NS * 8)
ROWS_PER_TILE = ACC_ROWS // NS


def _sc_aggregate():
    mesh = plsc.VectorSubcoreMesh(
        core_axis_name="c", subcore_axis_name="s", num_cores=NC, num_subcores=NS)

    @functools.partial(
        pl.kernel,
        out_type=jax.ShapeDtypeStruct((NC, ACC_ROWS, WIDE), jnp.float32),
        mesh=mesh,
        scratch_types=[
            pltpu.VMEM((IB, CHUNK), jnp.int32),
            pltpu.VMEM((IB, CHUNK), jnp.int32),
            pltpu.VMEM((CHUNK, WIDE), jnp.float32),
            pltpu.VMEM((CHUNK, WIDE), jnp.float32),
            pltpu.VMEM_SHARED((ACC_ROWS, WIDE), jnp.float32),
            pltpu.SemaphoreType.DMA,
            pltpu.SemaphoreType.DMA,
        ],
        compiler_params=pltpu.CompilerParams(use_tc_tiling_on_sc=False),
    )
    def agg(feat_hbm, src_hbm, dst_hbm, zeros_hbm, out_hbm,
            sidx, didx, rows0, rows1, acc, sem0, sem1):
        c = lax.axis_index("c")
        s = lax.axis_index("s")
        wid = s * NC + c
        pltpu.sync_copy(zeros_hbm, acc.at[pl.ds(s * ROWS_PER_TILE, ROWS_PER_TILE)])
        plsc.subcore_barrier()

        base = wid * EDGES_PER_W
        rows = (rows0, rows1)
        sems = (sem0, sem1)

        def body(b, _):
            blk = base // CHUNK + b * IB
            pltpu.sync_copy(src_hbm.at[pl.ds(blk, IB)], sidx)
            pltpu.sync_copy(dst_hbm.at[pl.ds(blk, IB)], didx)
            g = pltpu.async_copy(feat_hbm.at[sidx.at[0]], rows[0], sems[0])
            for j in range(IB):
                if j + 1 < IB:
                    g_next = pltpu.async_copy(
                        feat_hbm.at[sidx.at[j + 1]], rows[(j + 1) % 2],
                        sems[(j + 1) % 2])
                g.wait()
                pltpu.sync_copy(rows[j % 2], acc.at[didx.at[j]], add=True)
                if j + 1 < IB:
                    g = g_next
            return ()

        lax.fori_loop(0, BLOCKS_PER_W, body, ())
        plsc.subcore_barrier()
        pltpu.sync_copy(
            acc.at[pl.ds(s * ROWS_PER_TILE, ROWS_PER_TILE)],
            out_hbm.at[c].at[pl.ds(s * ROWS_PER_TILE, ROWS_PER_TILE)],
        )

    return agg


_SC_AGG = _sc_aggregate()

_BLK = 400
_GRID = N_NODES // _BLK


def _tc_body(feat_ref, p0_ref, p1_ref, wn_ref, ws_ref, b_ref, o_ref):
    acc = p0_ref[...] + p1_ref[...]
    nsum = acc[:, :D_IN]
    deg = acc[:, D_IN:D_IN + 1]
    scale = 1.0 / jnp.maximum(deg, 1.0)
    h = jnp.dot(feat_ref[...], ws_ref[...], preferred_element_type=jnp.float32)
    h = h + jnp.dot(nsum * scale, wn_ref[...], preferred_element_type=jnp.float32)
    o_ref[...] = h + b_ref[...]


def _tc_combine(feat, p0, p1, w_neigh, w_self, bias):
    return pl.pallas_call(
        _tc_body,
        grid=(_GRID,),
        in_specs=[
            pl.BlockSpec((_BLK, D_IN), lambda i: (i, 0)),
            pl.BlockSpec((_BLK, WIDE), lambda i: (i, 0)),
            pl.BlockSpec((_BLK, WIDE), lambda i: (i, 0)),
            pl.BlockSpec((D_IN, D_OUT), lambda i: (0, 0)),
            pl.BlockSpec((D_IN, D_OUT), lambda i: (0, 0)),
            pl.BlockSpec((1, D_OUT), lambda i: (0, 0)),
        ],
        out_specs=pl.BlockSpec((_BLK, D_OUT), lambda i: (i, 0)),
        out_shape=jax.ShapeDtypeStruct((N_NODES, D_OUT), jnp.float32),
    )(feat, p0, p1, w_neigh, w_self, bias)


def kernel(feat, edge_index, W_neigh, b_neigh, W_self, b_self):
    tail = jnp.zeros((1, WIDE - D_IN), jnp.float32).at[0, 0].set(1.0)
    feat_aug = jnp.concatenate(
        [feat, jnp.broadcast_to(tail, (N_NODES, WIDE - D_IN))], axis=1)

    pad = E_PAD - N_EDGES
    spread = jnp.arange(pad, dtype=jnp.int32)
    src = jnp.concatenate([edge_index[0], spread % N_NODES]).reshape(-1, CHUNK)
    dst = jnp.concatenate(
        [edge_index[1], N_NODES + spread % (ACC_ROWS - N_NODES)]).reshape(-1, CHUNK)

    zeros = jnp.zeros((ROWS_PER_TILE, WIDE), dtype=jnp.float32)
    parts = _SC_AGG(feat_aug, src, dst, zeros)
    bias = (b_neigh + b_self).reshape(1, D_OUT)
    return _tc_combine(feat, parts[0], parts[1], W_neigh, W_self, bias)


# speedup vs baseline: 10.4349x; 1.2326x over previous
import functools

import jax
import jax.numpy as jnp
from jax import lax
from jax.experimental import pallas as pl
from jax.experimental.pallas import tpu as pltpu
from jax.experimental.pallas import tpu_sc as plsc

N_NODES = 10000
N_EDGES = 320000
D_IN = 128
D_OUT = 128

NC = 2
NS = 16
NW = NC * NS
LANES = 16

CHUNK = 128
IB = 8
E_PAD = ((N_EDGES + NW * CHUNK * IB - 1) // (NW * CHUNK * IB)) * (NW * CHUNK * IB)
EDGES_PER_W = E_PAD // NW
CHUNKS_PER_W = EDGES_PER_W // CHUNK
BLOCKS_PER_W = CHUNKS_PER_W // IB
ACC_ROWS = 10240
ROWS_PER_TILE = ACC_ROWS // NS
DEG_ROWS = ACC_ROWS // LANES
DEG_STREAMS = DEG_ROWS // CHUNK


def _sc_aggregate():
    mesh = plsc.VectorSubcoreMesh(
        core_axis_name="c", subcore_axis_name="s", num_cores=NC, num_subcores=NS)

    @functools.partial(
        pl.kernel,
        out_type=(
            jax.ShapeDtypeStruct((NC, ACC_ROWS, D_IN), jnp.float32),
            jax.ShapeDtypeStruct((NC, DEG_ROWS, LANES), jnp.float32),
        ),
        mesh=mesh,
        scratch_types=[
            pltpu.VMEM((IB, CHUNK), jnp.int32),
            pltpu.VMEM((IB, CHUNK), jnp.int32),
            pltpu.VMEM((CHUNK, D_IN), jnp.float32),
            pltpu.VMEM((CHUNK, D_IN), jnp.float32),
            pltpu.VMEM((DEG_ROWS, LANES), jnp.float32),
            pltpu.VMEM((DEG_STREAMS, CHUNK), jnp.int32),
            pltpu.VMEM_SHARED((ACC_ROWS, D_IN), jnp.float32),
            pltpu.VMEM_SHARED((DEG_ROWS, LANES), jnp.float32),
            pltpu.SemaphoreType.DMA,
            pltpu.SemaphoreType.DMA,
        ],
        compiler_params=pltpu.CompilerParams(
            use_tc_tiling_on_sc=False, needs_layout_passes=False),
    )
    def agg(feat_hbm, src_hbm, dst_hbm, zeros_hbm, zeros16_hbm, iota_hbm,
            out_hbm, deg_out_hbm,
            sidx, didx, rows0, rows1, deg, iota_v, acc, acc_deg, sem0, sem1):
        c = lax.axis_index("c")
        s = lax.axis_index("s")
        wid = s * NC + c
        pltpu.sync_copy(zeros_hbm, acc.at[pl.ds(s * ROWS_PER_TILE, ROWS_PER_TILE)])
        pltpu.sync_copy(zeros16_hbm.at[pl.ds(0, DEG_ROWS // NS)],
                        acc_deg.at[pl.ds(s * (DEG_ROWS // NS), DEG_ROWS // NS)])
        pltpu.sync_copy(zeros16_hbm, deg)
        pltpu.sync_copy(iota_hbm, iota_v)
        plsc.subcore_barrier()

        base = wid * EDGES_PER_W
        rows = (rows0, rows1)
        sems = (sem0, sem1)

        def body(b, _):
            blk = base // CHUNK + b * IB
            pltpu.sync_copy(src_hbm.at[pl.ds(blk, IB)], sidx)
            pltpu.sync_copy(dst_hbm.at[pl.ds(blk, IB)], didx)
            g = pltpu.async_copy(feat_hbm.at[sidx.at[0]], rows[0], sems[0])
            for j in range(IB):
                if j + 1 < IB:
                    g_next = pltpu.async_copy(
                        feat_hbm.at[sidx.at[j + 1]], rows[(j + 1) % 2],
                        sems[(j + 1) % 2])
                for v in range(CHUNK // LANES):
                    d16 = didx[j, pl.ds(v * LANES, LANES)]
                    cnt, last = plsc.scan_count(d16)
                    plsc.addupdate_scatter(
                        deg,
                        [lax.shift_right_logical(d16, 4),
                         lax.bitwise_and(d16, 15)],
                        cnt.astype(jnp.float32), mask=last)
                g.wait()
                pltpu.sync_copy(rows[j % 2], acc.at[didx.at[j]], add=True)
                if j + 1 < IB:
                    g = g_next
            return ()

        lax.fori_loop(0, BLOCKS_PER_W, body, ())
        for t in range(DEG_STREAMS):
            pltpu.sync_copy(deg.at[pl.ds(t * CHUNK, CHUNK)],
                            acc_deg.at[iota_v.at[t]], add=True)
        plsc.subcore_barrier()
        pltpu.sync_copy(
            acc.at[pl.ds(s * ROWS_PER_TILE, ROWS_PER_TILE)],
            out_hbm.at[c].at[pl.ds(s * ROWS_PER_TILE, ROWS_PER_TILE)],
        )
        pltpu.sync_copy(
            acc_deg.at[pl.ds(s * (DEG_ROWS // NS), DEG_ROWS // NS)],
            deg_out_hbm.at[c].at[pl.ds(s * (DEG_ROWS // NS), DEG_ROWS // NS)],
        )

    return agg


_SC_AGG = _sc_aggregate()

_BLK = 400
_GRID = N_NODES // _BLK


def _tc_body(feat_ref, p0_ref, p1_ref, deg_ref, wn_ref, ws_ref, b_ref, o_ref):
    nsum = p0_ref[...] + p1_ref[...]
    scale = 1.0 / jnp.maximum(deg_ref[...], 1.0)
    h = jnp.dot(feat_ref[...], ws_ref[...], preferred_element_type=jnp.float32)
    h = h + jnp.dot(nsum * scale, wn_ref[...], preferred_element_type=jnp.float32)
    o_ref[...] = h + b_ref[...]


def _tc_combine(feat, p0, p1, deg, w_neigh, w_self, bias):
    return pl.pallas_call(
        _tc_body,
        grid=(_GRID,),
        in_specs=[
            pl.BlockSpec((_BLK, D_IN), lambda i: (i, 0)),
            pl.BlockSpec((_BLK, D_IN), lambda i: (i, 0)),
            pl.BlockSpec((_BLK, D_IN), lambda i: (i, 0)),
            pl.BlockSpec((_BLK, 1), lambda i: (i, 0)),
            pl.BlockSpec((D_IN, D_OUT), lambda i: (0, 0)),
            pl.BlockSpec((D_IN, D_OUT), lambda i: (0, 0)),
            pl.BlockSpec((1, D_OUT), lambda i: (0, 0)),
        ],
        out_specs=pl.BlockSpec((_BLK, D_OUT), lambda i: (i, 0)),
        out_shape=jax.ShapeDtypeStruct((N_NODES, D_OUT), jnp.float32),
    )(feat, p0, p1, deg, w_neigh, w_self, bias)


def kernel(feat, edge_index, W_neigh, b_neigh, W_self, b_self):
    pad = E_PAD - N_EDGES
    spread = jnp.arange(pad, dtype=jnp.int32)
    src = jnp.concatenate([edge_index[0], spread % N_NODES]).reshape(-1, CHUNK)
    dst = jnp.concatenate(
        [edge_index[1], N_NODES + spread % (ACC_ROWS - N_NODES)]).reshape(-1, CHUNK)

    zeros = jnp.zeros((ROWS_PER_TILE, D_IN), dtype=jnp.float32)
    zeros16 = jnp.zeros((DEG_ROWS, LANES), dtype=jnp.float32)
    iota = jnp.arange(DEG_ROWS, dtype=jnp.int32).reshape(DEG_STREAMS, CHUNK)
    parts, deg_parts = _SC_AGG(feat, src, dst, zeros, zeros16, iota)
    deg = (deg_parts[0] + deg_parts[1]).reshape(ACC_ROWS, 1)
    bias = (b_neigh + b_self).reshape(1, D_OUT)
    return _tc_combine(feat, parts[0], parts[1], deg[:N_NODES], W_neigh,
                       W_self, bias)


# speedup vs baseline: 13.0124x; 1.2470x over previous
import functools

import jax
import jax.numpy as jnp
from jax import lax
from jax.experimental import pallas as pl
from jax.experimental.pallas import tpu as pltpu
from jax.experimental.pallas import tpu_sc as plsc

N_NODES = 10000
N_EDGES = 320000
D_IN = 128
D_OUT = 128

NC = 2
NS = 16
NW = NC * NS
LANES = 16

CHUNK = 128
IB = 16
NBUF = 4
E_PAD = ((N_EDGES + NW * CHUNK * IB - 1) // (NW * CHUNK * IB)) * (NW * CHUNK * IB)
EDGES_PER_W = E_PAD // NW
CHUNKS_PER_W = EDGES_PER_W // CHUNK
BLOCKS_PER_W = CHUNKS_PER_W // IB
ACC_ROWS = 10240
ROWS_PER_TILE = ACC_ROWS // NS
DEG_ROWS = ACC_ROWS // LANES
DEG_STREAMS = DEG_ROWS // CHUNK


def _sc_aggregate():
    mesh = plsc.VectorSubcoreMesh(
        core_axis_name="c", subcore_axis_name="s", num_cores=NC, num_subcores=NS)

    @functools.partial(
        pl.kernel,
        out_type=(
            jax.ShapeDtypeStruct((NC, ACC_ROWS, D_IN), jnp.bfloat16),
            jax.ShapeDtypeStruct((NC, DEG_ROWS, LANES), jnp.float32),
        ),
        mesh=mesh,
        scratch_types=[
            pltpu.VMEM((IB, CHUNK), jnp.int32),
            pltpu.VMEM((IB, CHUNK), jnp.int32),
            [pltpu.VMEM((CHUNK, D_IN), jnp.bfloat16) for _ in range(NBUF)],
            pltpu.VMEM((DEG_ROWS, LANES), jnp.float32),
            pltpu.VMEM_SHARED((ACC_ROWS, D_IN), jnp.bfloat16),
            pltpu.VMEM_SHARED((DEG_ROWS, LANES), jnp.float32),
            [pltpu.SemaphoreType.DMA for _ in range(NBUF)],
        ],
        compiler_params=pltpu.CompilerParams(
            use_tc_tiling_on_sc=False, needs_layout_passes=False),
    )
    def agg(feat_hbm, src_hbm, dst_hbm, zerosb_hbm, zeros16_hbm,
            out_hbm, deg_out_hbm,
            sidx, didx, rows, deg, acc, acc_deg, sems):
        c = lax.axis_index("c")
        s = lax.axis_index("s")
        wid = s * NC + c
        pltpu.sync_copy(zerosb_hbm, acc.at[pl.ds(s * ROWS_PER_TILE, ROWS_PER_TILE)])
        pltpu.sync_copy(zeros16_hbm.at[pl.ds(0, DEG_ROWS // NS)],
                        acc_deg.at[pl.ds(s * (DEG_ROWS // NS), DEG_ROWS // NS)])
        pltpu.sync_copy(zeros16_hbm, deg)
        plsc.subcore_barrier()

        base_blk = wid * CHUNKS_PER_W

        def body(b, _):
            blk = base_blk + b * IB
            pltpu.sync_copy(src_hbm.at[pl.ds(blk, IB)], sidx)
            pltpu.sync_copy(dst_hbm.at[pl.ds(blk, IB)], didx)
            g = [pltpu.async_copy(feat_hbm.at[sidx.at[r]], rows[r], sems[r])
                 for r in range(NBUF)]
            for j in range(IB):
                r = j % NBUF
                g[r].wait()
                for v in range(CHUNK // LANES):
                    d16 = didx[j, pl.ds(v * LANES, LANES)]
                    cnt, last = plsc.scan_count(d16)
                    plsc.addupdate_scatter(
                        deg,
                        [lax.shift_right_logical(d16, 4),
                         lax.bitwise_and(d16, 15)],
                        cnt.astype(jnp.float32), mask=last)
                pltpu.sync_copy(rows[r], acc.at[didx.at[j]], add=True)
                if j + NBUF < IB:
                    g[r] = pltpu.async_copy(
                        feat_hbm.at[sidx.at[j + NBUF]], rows[r], sems[r])
            return ()

        lax.fori_loop(0, BLOCKS_PER_W, body, ())

        for t in range(DEG_STREAMS):
            for v in range(CHUNK // LANES):
                sidx[t, pl.ds(v * LANES, LANES)] = (
                    lax.iota(jnp.int32, LANES) + (t * CHUNK + v * LANES))
        for t in range(DEG_STREAMS):
            pltpu.sync_copy(deg.at[pl.ds(t * CHUNK, CHUNK)],
                            acc_deg.at[sidx.at[t]], add=True)
        plsc.subcore_barrier()
        pltpu.sync_copy(
            acc.at[pl.ds(s * ROWS_PER_TILE, ROWS_PER_TILE)],
            out_hbm.at[c].at[pl.ds(s * ROWS_PER_TILE, ROWS_PER_TILE)],
        )
        pltpu.sync_copy(
            acc_deg.at[pl.ds(s * (DEG_ROWS // NS), DEG_ROWS // NS)],
            deg_out_hbm.at[c].at[pl.ds(s * (DEG_ROWS // NS), DEG_ROWS // NS)],
        )

    return agg


_SC_AGG = _sc_aggregate()

_BLK = 400
_GRID = N_NODES // _BLK


def _tc_body(feat_ref, p0_ref, p1_ref, deg_ref, wn_ref, ws_ref, b_ref, o_ref):
    nsum = p0_ref[...].astype(jnp.float32) + p1_ref[...].astype(jnp.float32)
    scale = 1.0 / jnp.maximum(deg_ref[...], 1.0)
    h = jnp.dot(feat_ref[...], ws_ref[...], preferred_element_type=jnp.float32)
    h = h + jnp.dot(nsum * scale, wn_ref[...], preferred_element_type=jnp.float32)
    o_ref[...] = h + b_ref[...]


def _tc_combine(feat, p0, p1, deg, w_neigh, w_self, bias):
    return pl.pallas_call(
        _tc_body,
        grid=(_GRID,),
        in_specs=[
            pl.BlockSpec((_BLK, D_IN), lambda i: (i, 0)),
            pl.BlockSpec((_BLK, D_IN), lambda i: (i, 0)),
            pl.BlockSpec((_BLK, D_IN), lambda i: (i, 0)),
            pl.BlockSpec((_BLK, 1), lambda i: (i, 0)),
            pl.BlockSpec((D_IN, D_OUT), lambda i: (0, 0)),
            pl.BlockSpec((D_IN, D_OUT), lambda i: (0, 0)),
            pl.BlockSpec((1, D_OUT), lambda i: (0, 0)),
        ],
        out_specs=pl.BlockSpec((_BLK, D_OUT), lambda i: (i, 0)),
        out_shape=jax.ShapeDtypeStruct((N_NODES, D_OUT), jnp.float32),
    )(feat, p0, p1, deg, w_neigh, w_self, bias)


def kernel(feat, edge_index, W_neigh, b_neigh, W_self, b_self):
    feat_bf = feat.astype(jnp.bfloat16)
    pad = E_PAD - N_EDGES
    spread = jnp.arange(pad, dtype=jnp.int32)
    src = jnp.concatenate([edge_index[0], spread % N_NODES]).reshape(-1, CHUNK)
    dst = jnp.concatenate(
        [edge_index[1], N_NODES + spread % (ACC_ROWS - N_NODES)]).reshape(-1, CHUNK)

    zerosb = jnp.zeros((ROWS_PER_TILE, D_IN), dtype=jnp.bfloat16)
    zeros16 = jnp.zeros((DEG_ROWS, LANES), dtype=jnp.float32)
    parts, deg_parts = _SC_AGG(feat_bf, src, dst, zerosb, zeros16)
    deg = (deg_parts[0] + deg_parts[1]).reshape(ACC_ROWS, 1)
    bias = (b_neigh + b_self).reshape(1, D_OUT)
    return _tc_combine(feat, parts[0], parts[1], deg[:N_NODES], W_neigh,
                       W_self, bias)


# speedup vs baseline: 14.3597x; 1.1035x over previous
import functools

import jax
import jax.numpy as jnp
from jax import lax
from jax.experimental import pallas as pl
from jax.experimental.pallas import tpu as pltpu
from jax.experimental.pallas import tpu_sc as plsc

N_NODES = 10000
N_EDGES = 320000
D_IN = 128
D_OUT = 128

NC = 2
NS = 16
NW = NC * NS
LANES = 16

CHUNK = 128
IB = 16
N_CHUNKS = N_EDGES // CHUNK
N_FULL_BLOCKS = N_CHUNKS // IB
TAIL_CHUNKS = N_CHUNKS - N_FULL_BLOCKS * IB
HI_W = NW - TAIL_CHUNKS
BLK_LO = (N_FULL_BLOCKS + NW - 1) // NW
BLK_HI = BLK_LO - 1
NBUF = 4
ACC_ROWS = 10240
ROWS_PER_TILE = ACC_ROWS // NS
DEG_ROWS = ACC_ROWS // LANES
DEG_STREAMS = DEG_ROWS // CHUNK


def _sc_aggregate():
    mesh = plsc.VectorSubcoreMesh(
        core_axis_name="c", subcore_axis_name="s", num_cores=NC, num_subcores=NS)

    @functools.partial(
        pl.kernel,
        out_type=(
            jax.ShapeDtypeStruct((NC, ACC_ROWS, D_IN), jnp.bfloat16),
            jax.ShapeDtypeStruct((NC, DEG_ROWS, LANES), jnp.float32),
        ),
        mesh=mesh,
        scratch_types=[
            pltpu.VMEM((IB, CHUNK), jnp.int32),
            pltpu.VMEM((IB, CHUNK), jnp.int32),
            [pltpu.VMEM((CHUNK, D_IN), jnp.bfloat16) for _ in range(NBUF)],
            pltpu.VMEM((DEG_ROWS, LANES), jnp.float32),
            pltpu.VMEM_SHARED((ACC_ROWS, D_IN), jnp.bfloat16),
            pltpu.VMEM_SHARED((DEG_ROWS, LANES), jnp.float32),
            [pltpu.SemaphoreType.DMA for _ in range(NBUF)],
        ],
        compiler_params=pltpu.CompilerParams(
            use_tc_tiling_on_sc=False, needs_layout_passes=False),
    )
    def agg(feat_hbm, edges_hbm, zerosb_hbm, zeros16_hbm,
            out_hbm, deg_out_hbm,
            sidx, didx, rows, deg, acc, acc_deg, sems):
        c = lax.axis_index("c")
        s = lax.axis_index("s")
        wid = s * NC + c
        pltpu.sync_copy(zerosb_hbm, acc.at[pl.ds(s * ROWS_PER_TILE, ROWS_PER_TILE)])
        pltpu.sync_copy(zeros16_hbm.at[pl.ds(0, DEG_ROWS // NS)],
                        acc_deg.at[pl.ds(s * (DEG_ROWS // NS), DEG_ROWS // NS)])
        pltpu.sync_copy(zeros16_hbm, deg)
        plsc.subcore_barrier()

        def deg_hist(jrow):
            for v in range(CHUNK // LANES):
                d16 = didx[jrow, pl.ds(v * LANES, LANES)]
                cnt, last = plsc.scan_count(d16)
                plsc.addupdate_scatter(
                    deg,
                    [lax.shift_right_logical(d16, 4),
                     lax.bitwise_and(d16, 15)],
                    cnt.astype(jnp.float32), mask=last)

        base_blk = lax.select(wid < HI_W, BLK_LO * wid,
                              N_FULL_BLOCKS - BLK_HI * (NW - wid))
        n_blocks = lax.select(wid < HI_W, BLK_LO, BLK_HI)

        def body(b, _):
            chunk0 = (base_blk + b) * IB
            pltpu.sync_copy(edges_hbm.at[pl.ds(chunk0, IB)], sidx)
            pltpu.sync_copy(edges_hbm.at[pl.ds(N_CHUNKS + chunk0, IB)], didx)
            g = [pltpu.async_copy(feat_hbm.at[sidx.at[r]], rows[r], sems[r])
                 for r in range(NBUF)]
            for j in range(IB):
                r = j % NBUF
                g[r].wait()
                deg_hist(j)
                pltpu.sync_copy(rows[r], acc.at[didx.at[j]], add=True)
                if j + NBUF < IB:
                    g[r] = pltpu.async_copy(
                        feat_hbm.at[sidx.at[j + NBUF]], rows[r], sems[r])
            return ()

        lax.fori_loop(0, n_blocks, body, ())

        @pl.when(wid >= HI_W)
        def _():
            ctail = N_FULL_BLOCKS * IB + (wid - HI_W)
            pltpu.sync_copy(edges_hbm.at[pl.ds(ctail, 1)],
                            sidx.at[pl.ds(0, 1)])
            pltpu.sync_copy(edges_hbm.at[pl.ds(N_CHUNKS + ctail, 1)],
                            didx.at[pl.ds(0, 1)])
            pltpu.async_copy(feat_hbm.at[sidx.at[0]], rows[0], sems[0]).wait()
            deg_hist(0)
            pltpu.sync_copy(rows[0], acc.at[didx.at[0]], add=True)

        for t in range(DEG_STREAMS):
            for v in range(CHUNK // LANES):
                sidx[t, pl.ds(v * LANES, LANES)] = (
                    lax.iota(jnp.int32, LANES) + (t * CHUNK + v * LANES))
        for t in range(DEG_STREAMS):
            pltpu.sync_copy(deg.at[pl.ds(t * CHUNK, CHUNK)],
                            acc_deg.at[sidx.at[t]], add=True)
        plsc.subcore_barrier()
        pltpu.sync_copy(
            acc.at[pl.ds(s * ROWS_PER_TILE, ROWS_PER_TILE)],
            out_hbm.at[c].at[pl.ds(s * ROWS_PER_TILE, ROWS_PER_TILE)],
        )
        pltpu.sync_copy(
            acc_deg.at[pl.ds(s * (DEG_ROWS // NS), DEG_ROWS // NS)],
            deg_out_hbm.at[c].at[pl.ds(s * (DEG_ROWS // NS), DEG_ROWS // NS)],
        )

    return agg


_SC_AGG = _sc_aggregate()

_BLK = 512
_GRID = (N_NODES + _BLK - 1) // _BLK
_SUB = _BLK // LANES


def _tc_body(feat_ref, p0_ref, p1_ref, deg_ref, wn_ref, ws_ref, bn_ref,
             bs_ref, o_ref):
    nsum = p0_ref[...].astype(jnp.float32) + p1_ref[...].astype(jnp.float32)
    dsum = deg_ref[0] + deg_ref[1]
    scale = (1.0 / jnp.maximum(dsum, 1.0))[:, :, None]
    nmean = (nsum.reshape(_SUB, LANES, D_IN) * scale).reshape(_BLK, D_IN)
    h = jnp.dot(feat_ref[...], ws_ref[...], preferred_element_type=jnp.float32)
    h = h + jnp.dot(nmean, wn_ref[...], preferred_element_type=jnp.float32)
    o_ref[...] = h + (bn_ref[...] + bs_ref[...])


def _tc_combine(feat, p0, p1, deg_parts, w_neigh, w_self, b_neigh, b_self):
    return pl.pallas_call(
        _tc_body,
        grid=(_GRID,),
        in_specs=[
            pl.BlockSpec((_BLK, D_IN), lambda i: (i, 0)),
            pl.BlockSpec((_BLK, D_IN), lambda i: (i, 0)),
            pl.BlockSpec((_BLK, D_IN), lambda i: (i, 0)),
            pl.BlockSpec((NC, _SUB, LANES), lambda i: (0, i, 0)),
            pl.BlockSpec((D_IN, D_OUT), lambda i: (0, 0)),
            pl.BlockSpec((D_IN, D_OUT), lambda i: (0, 0)),
            pl.BlockSpec((1, D_OUT), lambda i: (0, 0)),
            pl.BlockSpec((1, D_OUT), lambda i: (0, 0)),
        ],
        out_specs=pl.BlockSpec((_BLK, D_OUT), lambda i: (i, 0)),
        out_shape=jax.ShapeDtypeStruct((N_NODES, D_OUT), jnp.float32),
    )(feat, p0, p1, deg_parts, w_neigh, w_self, b_neigh, b_self)


def kernel(feat, edge_index, W_neigh, b_neigh, W_self, b_self):
    feat_bf = feat.astype(jnp.bfloat16)
    edges = edge_index.reshape(2 * N_CHUNKS, CHUNK)

    zerosb = jnp.zeros((ROWS_PER_TILE, D_IN), dtype=jnp.bfloat16)
    zeros16 = jnp.zeros((DEG_ROWS, LANES), dtype=jnp.float32)
    parts, deg_parts = _SC_AGG(feat_bf, edges, zerosb, zeros16)
    return _tc_combine(feat, parts[0], parts[1], deg_parts, W_neigh, W_self,
                       b_neigh.reshape(1, D_OUT), b_self.reshape(1, D_OUT))


# speedup vs baseline: 14.3728x; 1.0009x over previous
import functools

import jax
import jax.numpy as jnp
from jax import lax
from jax.experimental import pallas as pl
from jax.experimental.pallas import tpu as pltpu
from jax.experimental.pallas import tpu_sc as plsc

N_NODES = 10000
N_EDGES = 320000
D_IN = 128
D_OUT = 128

NC = 2
NS = 16
NW = NC * NS
LANES = 16

CHUNK = 128
IB = 16
N_CHUNKS = N_EDGES // CHUNK
N_FULL_BLOCKS = N_CHUNKS // IB
TAIL_CHUNKS = N_CHUNKS - N_FULL_BLOCKS * IB
HI_W = NW - TAIL_CHUNKS
BLK_LO = (N_FULL_BLOCKS + NW - 1) // NW
BLK_HI = BLK_LO - 1
NBUF = 4
ACC_ROWS = 10240
ROWS_PER_TILE = ACC_ROWS // NS
DEG_ROWS = ACC_ROWS // LANES
DEG_STREAMS = DEG_ROWS // CHUNK


def _sc_aggregate():
    mesh = plsc.VectorSubcoreMesh(
        core_axis_name="c", subcore_axis_name="s", num_cores=NC, num_subcores=NS)

    @functools.partial(
        pl.kernel,
        out_type=(
            jax.ShapeDtypeStruct((NC, ACC_ROWS, D_IN), jnp.bfloat16),
            jax.ShapeDtypeStruct((NC, DEG_ROWS, LANES), jnp.float32),
        ),
        mesh=mesh,
        scratch_types=[
            pltpu.VMEM((IB, CHUNK), jnp.int32),
            pltpu.VMEM((IB, CHUNK), jnp.int32),
            [pltpu.VMEM((CHUNK, D_IN), jnp.bfloat16) for _ in range(NBUF)],
            pltpu.VMEM((DEG_ROWS, LANES), jnp.float32),
            pltpu.VMEM_SHARED((ACC_ROWS, D_IN), jnp.bfloat16),
            pltpu.VMEM_SHARED((DEG_ROWS, LANES), jnp.float32),
            [pltpu.SemaphoreType.DMA for _ in range(NBUF)],
        ],
        compiler_params=pltpu.CompilerParams(
            use_tc_tiling_on_sc=False, needs_layout_passes=False),
    )
    def agg(feat_hbm, edges_hbm, zerosb_hbm, zeros16_hbm,
            out_hbm, deg_out_hbm,
            sidx, didx, rows, deg, acc, acc_deg, sems):
        c = lax.axis_index("c")
        s = lax.axis_index("s")
        wid = s * NC + c
        pltpu.sync_copy(zerosb_hbm, acc.at[pl.ds(s * ROWS_PER_TILE, ROWS_PER_TILE)])
        pltpu.sync_copy(zeros16_hbm.at[pl.ds(0, DEG_ROWS // NS)],
                        acc_deg.at[pl.ds(s * (DEG_ROWS // NS), DEG_ROWS // NS)])
        pltpu.sync_copy(zeros16_hbm, deg)
        plsc.subcore_barrier()

        def deg_hist(jrow):
            for v in range(CHUNK // LANES):
                d16 = didx[jrow, pl.ds(v * LANES, LANES)]
                cnt, last = plsc.scan_count(d16)
                plsc.addupdate_scatter(
                    deg,
                    [lax.shift_right_logical(d16, 4),
                     lax.bitwise_and(d16, 15)],
                    cnt.astype(jnp.float32), mask=last)

        base_blk = lax.select(wid < HI_W, BLK_LO * wid,
                              N_FULL_BLOCKS - BLK_HI * (NW - wid))
        n_blocks = lax.select(wid < HI_W, BLK_LO, BLK_HI)

        def body(b, _):
            chunk0 = (base_blk + b) * IB
            pltpu.sync_copy(edges_hbm.at[pl.ds(chunk0, IB)], sidx)
            pltpu.sync_copy(edges_hbm.at[pl.ds(N_CHUNKS + chunk0, IB)], didx)
            g = [pltpu.async_copy(feat_hbm.at[sidx.at[r]], rows[r], sems[r])
                 for r in range(NBUF)]
            for j in range(IB):
                r = j % NBUF
                g[r].wait()
                deg_hist(j)
                pltpu.sync_copy(rows[r], acc.at[didx.at[j]], add=True)
                if j + NBUF < IB:
                    g[r] = pltpu.async_copy(
                        feat_hbm.at[sidx.at[j + NBUF]], rows[r], sems[r])
            return ()

        lax.fori_loop(0, n_blocks, body, ())

        @pl.when(wid >= HI_W)
        def _():
            ctail = N_FULL_BLOCKS * IB + (wid - HI_W)
            pltpu.sync_copy(edges_hbm.at[pl.ds(ctail, 1)],
                            sidx.at[pl.ds(0, 1)])
            pltpu.sync_copy(edges_hbm.at[pl.ds(N_CHUNKS + ctail, 1)],
                            didx.at[pl.ds(0, 1)])
            pltpu.async_copy(feat_hbm.at[sidx.at[0]], rows[0], sems[0]).wait()
            deg_hist(0)
            pltpu.sync_copy(rows[0], acc.at[didx.at[0]], add=True)

        for t in range(DEG_STREAMS):
            for v in range(CHUNK // LANES):
                sidx[t, pl.ds(v * LANES, LANES)] = (
                    lax.iota(jnp.int32, LANES) + (t * CHUNK + v * LANES))
        for t in range(DEG_STREAMS):
            pltpu.sync_copy(deg.at[pl.ds(t * CHUNK, CHUNK)],
                            acc_deg.at[sidx.at[t]], add=True)
        plsc.subcore_barrier()
        pltpu.sync_copy(
            acc.at[pl.ds(s * ROWS_PER_TILE, ROWS_PER_TILE)],
            out_hbm.at[c].at[pl.ds(s * ROWS_PER_TILE, ROWS_PER_TILE)],
        )
        pltpu.sync_copy(
            acc_deg.at[pl.ds(s * (DEG_ROWS // NS), DEG_ROWS // NS)],
            deg_out_hbm.at[c].at[pl.ds(s * (DEG_ROWS // NS), DEG_ROWS // NS)],
        )

    return agg


_SC_AGG = _sc_aggregate()

_BLK = 512
_GRID = (N_NODES + _BLK - 1) // _BLK
_SUB = _BLK // LANES


def _tc_body(feat_ref, p_ref, deg_ref, wn_ref, ws_ref, bn_ref, bs_ref, o_ref):
    nsum = p_ref[0].astype(jnp.float32) + p_ref[1].astype(jnp.float32)
    dsum = deg_ref[0] + deg_ref[1]
    scale = (1.0 / jnp.maximum(dsum, 1.0))[:, :, None]
    nmean = (nsum.reshape(_SUB, LANES, D_IN) * scale).reshape(_BLK, D_IN)
    h = jnp.dot(feat_ref[...], ws_ref[...], preferred_element_type=jnp.float32)
    h = h + jnp.dot(nmean, wn_ref[...], preferred_element_type=jnp.float32)
    o_ref[...] = h + (bn_ref[...] + bs_ref[...])[None, :]


def _tc_combine(feat, parts, deg_parts, w_neigh, w_self, b_neigh, b_self):
    return pl.pallas_call(
        _tc_body,
        grid=(_GRID,),
        in_specs=[
            pl.BlockSpec((_BLK, D_IN), lambda i: (i, 0)),
            pl.BlockSpec((NC, _BLK, D_IN), lambda i: (0, i, 0)),
            pl.BlockSpec((NC, _SUB, LANES), lambda i: (0, i, 0)),
            pl.BlockSpec((D_IN, D_OUT), lambda i: (0, 0)),
            pl.BlockSpec((D_IN, D_OUT), lambda i: (0, 0)),
            pl.BlockSpec((D_OUT,), lambda i: (0,)),
            pl.BlockSpec((D_OUT,), lambda i: (0,)),
        ],
        out_specs=pl.BlockSpec((_BLK, D_OUT), lambda i: (i, 0)),
        out_shape=jax.ShapeDtypeStruct((N_NODES, D_OUT), jnp.float32),
    )(feat, parts, deg_parts, w_neigh, w_self, b_neigh, b_self)


def kernel(feat, edge_index, W_neigh, b_neigh, W_self, b_self):
    feat_bf = feat.astype(jnp.bfloat16)
    edges = edge_index.reshape(2 * N_CHUNKS, CHUNK)

    zerosb = jnp.zeros((ROWS_PER_TILE, D_IN), dtype=jnp.bfloat16)
    zeros16 = jnp.zeros((DEG_ROWS, LANES), dtype=jnp.float32)
    parts, deg_parts = _SC_AGG(feat_bf, edges, zerosb, zeros16)
    return _tc_combine(feat, parts, deg_parts, W_neigh, W_self, b_neigh, b_self)


# speedup vs baseline: 15.5316x; 1.0806x over previous
import functools

import jax
import jax.numpy as jnp
from jax import lax
from jax.experimental import pallas as pl
from jax.experimental.pallas import tpu as pltpu
from jax.experimental.pallas import tpu_sc as plsc

N_NODES = 10000
N_EDGES = 320000
D_IN = 128
D_OUT = 128

NC = 2
NS = 16
NW = NC * NS
LANES = 16

CHUNK = 128
IB = 16
N_CHUNKS = N_EDGES // CHUNK
N_FULL_BLOCKS = N_CHUNKS // IB
TAIL_CHUNKS = N_CHUNKS - N_FULL_BLOCKS * IB
HI_W = NW - TAIL_CHUNKS
BLK_LO = (N_FULL_BLOCKS + NW - 1) // NW
BLK_HI = BLK_LO - 1
NBUF = 4
FCHUNK = 128
ACC_ROWS = 10240
ROWS_PER_TILE = ACC_ROWS // NS
DEG_ROWS = ACC_ROWS // LANES
DEG_STREAMS = DEG_ROWS // CHUNK


def _sc_aggregate():
    mesh = plsc.VectorSubcoreMesh(
        core_axis_name="c", subcore_axis_name="s", num_cores=NC, num_subcores=NS)

    @functools.partial(
        pl.kernel,
        out_type=(
            jax.ShapeDtypeStruct((NC, ACC_ROWS, D_IN), jnp.float32),
            jax.ShapeDtypeStruct((NC, DEG_ROWS, LANES), jnp.float32),
        ),
        mesh=mesh,
        scratch_types=[
            [pltpu.VMEM((IB, CHUNK), jnp.int32) for _ in range(2)],
            [pltpu.VMEM((IB, CHUNK), jnp.int32) for _ in range(2)],
            [pltpu.VMEM((CHUNK, D_IN), jnp.bfloat16) for _ in range(NBUF)],
            pltpu.VMEM((FCHUNK, D_IN), jnp.float32),
            pltpu.VMEM((DEG_ROWS, LANES), jnp.float32),
            pltpu.VMEM_SHARED((ACC_ROWS, D_IN), jnp.bfloat16),
            pltpu.VMEM_SHARED((DEG_ROWS, LANES), jnp.float32),
            [pltpu.SemaphoreType.DMA for _ in range(NBUF)],
            [pltpu.SemaphoreType.DMA for _ in range(2)],
        ],
        compiler_params=pltpu.CompilerParams(
            use_tc_tiling_on_sc=False, needs_layout_passes=False),
    )
    def agg(feat_hbm, edges_hbm, zerosb_hbm, zeros16_hbm,
            out_hbm, deg_out_hbm,
            sidx, didx, rows, frow, deg, acc, acc_deg, sems, isems):
        c = lax.axis_index("c")
        s = lax.axis_index("s")
        wid = s * NC + c
        pltpu.sync_copy(zerosb_hbm, acc.at[pl.ds(s * ROWS_PER_TILE, ROWS_PER_TILE)])
        pltpu.sync_copy(zeros16_hbm.at[pl.ds(0, DEG_ROWS // NS)],
                        acc_deg.at[pl.ds(s * (DEG_ROWS // NS), DEG_ROWS // NS)])
        pltpu.sync_copy(zeros16_hbm, deg)
        plsc.subcore_barrier()

        def deg_hist(didx_p, jrow):
            for v in range(CHUNK // LANES):
                d16 = didx_p[jrow, pl.ds(v * LANES, LANES)]
                cnt, last = plsc.scan_count(d16)
                plsc.addupdate_scatter(
                    deg,
                    [lax.shift_right_logical(d16, 4),
                     lax.bitwise_and(d16, 15)],
                    cnt.astype(jnp.float32), mask=last)

        base_blk = lax.select(wid < HI_W, BLK_LO * wid,
                              N_FULL_BLOCKS - BLK_HI * (NW - wid))

        def load_idx(b, p):
            chunk0 = lax.min((base_blk + b) * IB, (N_FULL_BLOCKS - 1) * IB)
            a = pltpu.async_copy(edges_hbm.at[pl.ds(chunk0, IB)], sidx[p],
                                 isems[0])
            d = pltpu.async_copy(edges_hbm.at[pl.ds(N_CHUNKS + chunk0, IB)],
                                 didx[p], isems[1])
            return a, d

        def run_block(p):
            g = [pltpu.async_copy(feat_hbm.at[sidx[p].at[r]], rows[r], sems[r])
                 for r in range(NBUF)]
            for j in range(IB):
                r = j % NBUF
                g[r].wait()
                deg_hist(didx[p], j)
                pltpu.sync_copy(rows[r], acc.at[didx[p].at[j]], add=True)
                if j + NBUF < IB:
                    g[r] = pltpu.async_copy(
                        feat_hbm.at[sidx[p].at[j + NBUF]], rows[r], sems[r])

        ld = load_idx(0, 0)
        for b in range(BLK_HI):
            ld[0].wait()
            ld[1].wait()
            ld = load_idx(b + 1, (b + 1) % 2)
            run_block(b % 2)
        ld[0].wait()
        ld[1].wait()

        @pl.when(wid < HI_W)
        def _():
            run_block(BLK_HI % 2)

        @pl.when(wid >= HI_W)
        def _():
            ctail = N_FULL_BLOCKS * IB + (wid - HI_W)
            pltpu.sync_copy(edges_hbm.at[pl.ds(ctail, 1)],
                            sidx[0].at[pl.ds(0, 1)])
            pltpu.sync_copy(edges_hbm.at[pl.ds(N_CHUNKS + ctail, 1)],
                            didx[0].at[pl.ds(0, 1)])
            pltpu.async_copy(feat_hbm.at[sidx[0].at[0]], rows[0], sems[0]).wait()
            deg_hist(didx[0], 0)
            pltpu.sync_copy(rows[0], acc.at[didx[0].at[0]], add=True)

        for t in range(DEG_STREAMS):
            for v in range(CHUNK // LANES):
                sidx[0][t, pl.ds(v * LANES, LANES)] = (
                    lax.iota(jnp.int32, LANES) + (t * CHUNK + v * LANES))
        for t in range(DEG_STREAMS):
            pltpu.sync_copy(deg.at[pl.ds(t * CHUNK, CHUNK)],
                            acc_deg.at[sidx[0].at[t]], add=True)
        plsc.subcore_barrier()
        for t in range(ROWS_PER_TILE // FCHUNK):
            pltpu.sync_copy(
                acc.at[pl.ds(s * ROWS_PER_TILE + t * FCHUNK, FCHUNK)],
                rows[0].at[pl.ds(0, FCHUNK)])

            def conv_row(i, _):
                for grp in range(D_IN // 32):
                    u = plsc.bitcast(rows[0][i, pl.ds(grp * 32, 32)],
                                     jnp.uint32)
                    frow[i, pl.ds(grp * 32, LANES)] = plsc.bitcast(
                        u << 16, jnp.float32)
                    frow[i, pl.ds(grp * 32 + LANES, LANES)] = plsc.bitcast(
                        lax.bitwise_and(u, jnp.uint32(0xFFFF0000)),
                        jnp.float32)
                return ()

            lax.fori_loop(0, FCHUNK, conv_row, ())
            pltpu.sync_copy(
                frow,
                out_hbm.at[c].at[pl.ds(s * ROWS_PER_TILE + t * FCHUNK, FCHUNK)])
        pltpu.sync_copy(
            acc_deg.at[pl.ds(s * (DEG_ROWS // NS), DEG_ROWS // NS)],
            deg_out_hbm.at[c].at[pl.ds(s * (DEG_ROWS // NS), DEG_ROWS // NS)],
        )

    return agg


_SC_AGG = _sc_aggregate()

_BLK = 512
_GRID = (N_NODES + _BLK - 1) // _BLK
_SUB = _BLK // LANES


def _tc_body(feat_ref, p_ref, deg_ref, wn_ref, ws_ref, bn_ref, bs_ref, o_ref):
    nsum = p_ref[0] + p_ref[1]
    dsum = deg_ref[0] + deg_ref[1]
    scale = (1.0 / jnp.maximum(dsum, 1.0))[:, :, None]
    nmean = (nsum.reshape(_SUB, LANES, D_IN) * scale).reshape(_BLK, D_IN)
    h = jnp.dot(feat_ref[...], ws_ref[...], preferred_element_type=jnp.float32)
    h = h + jnp.dot(nmean, wn_ref[...], preferred_element_type=jnp.float32)
    o_ref[...] = h + (bn_ref[...] + bs_ref[...])[None, :]


def _tc_combine(feat, parts, deg_parts, w_neigh, w_self, b_neigh, b_self):
    return pl.pallas_call(
        _tc_body,
        grid=(_GRID,),
        in_specs=[
            pl.BlockSpec((_BLK, D_IN), lambda i: (i, 0)),
            pl.BlockSpec((NC, _BLK, D_IN), lambda i: (0, i, 0)),
            pl.BlockSpec((NC, _SUB, LANES), lambda i: (0, i, 0)),
            pl.BlockSpec((D_IN, D_OUT), lambda i: (0, 0)),
            pl.BlockSpec((D_IN, D_OUT), lambda i: (0, 0)),
            pl.BlockSpec((D_OUT,), lambda i: (0,)),
            pl.BlockSpec((D_OUT,), lambda i: (0,)),
        ],
        out_specs=pl.BlockSpec((_BLK, D_OUT), lambda i: (i, 0)),
        out_shape=jax.ShapeDtypeStruct((N_NODES, D_OUT), jnp.float32),
    )(feat, parts, deg_parts, w_neigh, w_self, b_neigh, b_self)


_PERM = [(p // 32) * 32 + 2 * (p % 16) + (p % 32) // 16 for p in range(D_IN)]


def kernel(feat, edge_index, W_neigh, b_neigh, W_self, b_self):
    feat_bf = feat.astype(jnp.bfloat16)
    edges = edge_index.reshape(2 * N_CHUNKS, CHUNK)
    w_neigh_p = W_neigh[jnp.array(_PERM, dtype=jnp.int32), :]

    zerosb = jnp.zeros((ROWS_PER_TILE, D_IN), dtype=jnp.bfloat16)
    zeros16 = jnp.zeros((DEG_ROWS, LANES), dtype=jnp.float32)
    parts, deg_parts = _SC_AGG(feat_bf, edges, zerosb, zeros16)
    return _tc_combine(feat, parts, deg_parts, w_neigh_p, W_self, b_neigh,
                       b_self)
